# Initial kernel scaffold; baseline (speedup 1.0000x reference)
import jax
import jax.numpy as jnp
from jax.experimental import pallas as pl


def kernel(words_seq, table):
    raise NotImplementedError("write your pallas kernel here")


# speedup vs baseline: 3.9572x; 3.9572x over previous
import functools

import jax
import jax.numpy as jnp
from jax import lax
from jax.experimental import pallas as pl
from jax.experimental.pallas import tpu as pltpu
from jax.experimental.pallas import tpu_sc as plsc

_D = 64
_CHUNK = 512


@functools.lru_cache(maxsize=None)
def _make_gather(n_rows: int, vocab: int, d: int):
    info = plsc.get_sparse_core_info()
    nw = info.num_cores * info.num_subcores
    rows_per_w = n_rows // nw
    n_chunks = rows_per_w // _CHUNK
    assert rows_per_w % _CHUNK == 0 and n_rows % nw == 0

    mesh = plsc.VectorSubcoreMesh(core_axis_name="c", subcore_axis_name="s")

    @functools.partial(
        pl.kernel,
        mesh=mesh,
        out_type=jax.ShapeDtypeStruct((n_rows, d), jnp.float32),
        scratch_types=[
            pltpu.VMEM((_CHUNK,), jnp.int32),
            pltpu.VMEM((_CHUNK, d), jnp.float32),
            pltpu.SemaphoreType.DMA,
        ],
        compiler_params=pltpu.CompilerParams(use_tc_tiling_on_sc=False),
    )
    def k(idx_hbm, table_hbm, out_hbm, idx_v, rows_v, sem):
        wid = lax.axis_index("s") * info.num_cores + lax.axis_index("c")
        base = wid * rows_per_w

        def body(g, carry):
            off = base + g * _CHUNK
            pltpu.sync_copy(idx_hbm.at[pl.ds(off, _CHUNK)], idx_v)
            pltpu.async_copy(table_hbm.at[idx_v], rows_v, sem).wait()
            pltpu.sync_copy(rows_v, out_hbm.at[pl.ds(off, _CHUNK)])
            return carry

        lax.fori_loop(0, n_chunks, body, 0)

    return k


def kernel(words_seq, table):
    b, s = words_seq.shape
    v, d = table.shape
    idx_flat = words_seq.reshape(-1).astype(jnp.int32)
    out = _make_gather(b * s, v, d)(idx_flat, table)
    return out.reshape(b, s, d)


# speedup vs baseline: 4.2494x; 1.0738x over previous
import functools

import jax
import jax.numpy as jnp
from jax import lax
from jax.experimental import pallas as pl
from jax.experimental.pallas import tpu as pltpu
from jax.experimental.pallas import tpu_sc as plsc

_CHUNK = 320
_NR = 4


@functools.lru_cache(maxsize=None)
def _make_gather(n_rows: int, vocab: int, d: int):
    info = plsc.get_sparse_core_info()
    nw = info.num_cores * info.num_subcores
    rows_per_w = n_rows // nw
    n_chunks = rows_per_w // _CHUNK
    assert n_rows % nw == 0 and rows_per_w % _CHUNK == 0
    steady = n_chunks - 8
    assert steady > 0 and steady % _NR == 0

    mesh = plsc.VectorSubcoreMesh(core_axis_name="c", subcore_axis_name="s")

    @functools.partial(
        pl.kernel,
        mesh=mesh,
        out_type=jax.ShapeDtypeStruct((n_rows, d), jnp.float32),
        scratch_types=[
            pltpu.VMEM((rows_per_w,), jnp.int32),
            *[pltpu.VMEM((_CHUNK, d), jnp.float32) for _ in range(_NR)],
            *[pltpu.SemaphoreType.DMA for _ in range(2 * _NR)],
        ],
        compiler_params=pltpu.CompilerParams(use_tc_tiling_on_sc=False),
    )
    def k(idx_hbm, table_hbm, out_hbm, idx_v, *bufs_and_sems):
        rows = bufs_and_sems[:_NR]
        gsem = bufs_and_sems[_NR:2 * _NR]
        osem = bufs_and_sems[2 * _NR:]
        wid = lax.axis_index("s") * info.num_cores + lax.axis_index("c")
        base = wid * rows_per_w

        pltpu.sync_copy(idx_hbm.at[pl.ds(base, rows_per_w)], idx_v)

        def gather_desc(g, slot):
            src = table_hbm.at[idx_v.at[pl.ds(g * _CHUNK, _CHUNK)]]
            return pltpu.make_async_copy(src, rows[slot], gsem[slot])

        def out_desc(g, slot):
            dst = out_hbm.at[pl.ds(base + g * _CHUNK, _CHUNK)]
            return pltpu.make_async_copy(rows[slot], dst, osem[slot])

        def sg(g, slot):
            gather_desc(g, slot).start()

        def wg(g, slot):
            gather_desc(g, slot).wait()

        def so(g, slot):
            out_desc(g, slot).start()

        def wo(g, slot):
            out_desc(g, slot).wait()

        sg(0, 0)
        sg(1, 1)
        sg(2, 2)
        sg(3, 3)
        wg(0, 0)
        so(0, 0)
        wo(0, 0)
        sg(4, 0)
        wg(1, 1)
        so(1, 1)
        wo(1, 1)
        sg(5, 1)
        wg(2, 2)
        so(2, 2)

        def body(t, carry):
            g0 = 6 + t * _NR
            for u in range(_NR):
                g = g0 + u
                slot = (u + 2) % _NR
                pslot = (u + 3) % _NR
                wo(g - 4, slot)
                sg(g, slot)
                wg(g - 3, pslot)
                so(g - 3, pslot)
            return carry

        lax.fori_loop(0, steady // _NR, body, 0)

        n = n_chunks
        for g in (n - 2, n - 1):
            slot = g % _NR
            pslot = (g - 3) % _NR
            wo(g - 4, slot)
            sg(g, slot)
            wg(g - 3, pslot)
            so(g - 3, pslot)
        for g in (n - 3, n - 2, n - 1):
            wg(g, g % _NR)
            so(g, g % _NR)
        for g in (n - 4, n - 3, n - 2, n - 1):
            wo(g, g % _NR)

    return k


def kernel(words_seq, table):
    b, s = words_seq.shape
    v, d = table.shape
    idx_flat = words_seq.reshape(-1).astype(jnp.int32)
    out = _make_gather(b * s, v, d)(idx_flat, table)
    return out.reshape(b, s, d)
